# Initial kernel scaffold; baseline (speedup 1.0000x reference)
import functools

import jax
import jax.numpy as jnp
from jax import lax
from jax.experimental import pallas as pl
from jax.experimental.pallas import tpu as pltpu
from jax.experimental.pallas import tpu_sc as plsc

_CHUNK = 128
_NBUF = 8


def _make_sc_gather(V, D, N):
    info = plsc.get_sparse_core_info()
    nw = info.num_cores * info.num_subcores
    rows_per_w = N // nw
    n_ch = rows_per_w // _CHUNK
    n_outer = n_ch // _NBUF
    assert rows_per_w % (_CHUNK * _NBUF) == 0
    mesh = plsc.VectorSubcoreMesh(core_axis_name="c", subcore_axis_name="s")

    @functools.partial(
        pl.kernel,
        mesh=mesh,
        out_type=jax.ShapeDtypeStruct((N, D), jnp.float32),
        scratch_types=[
            pltpu.VMEM((n_ch, _CHUNK), jnp.int32),
            pltpu.VMEM((_NBUF * _CHUNK, D), jnp.float32),
        ] + [pltpu.SemaphoreType.DMA] * _NBUF,
    )
    def gather(idx_hbm, table_hbm, out_hbm, idx_v, rows_v, *sems):
        wid = lax.axis_index("s") * info.num_cores + lax.axis_index("c")
        row_base = wid * rows_per_w
        pltpu.sync_copy(idx_hbm.at[pl.ds(wid * n_ch, n_ch)], idx_v)

        def body(g, carry):
            ch0 = g * _NBUF
            cps = []
            for bidx in range(_NBUF):
                cps.append(pltpu.async_copy(
                    table_hbm.at[idx_v.at[ch0 + bidx]],
                    rows_v.at[pl.ds(bidx * _CHUNK, _CHUNK)],
                    sems[bidx]))
            for cp in cps:
                cp.wait()
            pltpu.sync_copy(
                rows_v,
                out_hbm.at[pl.ds(row_base + ch0 * _CHUNK, _NBUF * _CHUNK)])
            return carry

        lax.fori_loop(0, n_outer, body, 0)

    return gather


def _make_tc_head(Bb, K, C, BB):
    def body(e_ref, w_ref, b_ref, o_ref):
        e = e_ref[...].astype(jnp.bfloat16)
        logits = lax.dot_general(e, w_ref[...], (((1,), (1,)), ((), ())),
                                 preferred_element_type=jnp.float32)
        logits = logits + b_ref[...]
        m = jnp.max(logits, axis=-1, keepdims=True)
        p = jnp.exp(logits - m)
        o_ref[...] = p / jnp.sum(p, axis=-1, keepdims=True)

    return pl.pallas_call(
        body,
        grid=(Bb // BB,),
        in_specs=[
            pl.BlockSpec((BB, K), lambda i: (i, 0)),
            pl.BlockSpec((C, K), lambda i: (0, 0)),
            pl.BlockSpec((1, C), lambda i: (0, 0)),
        ],
        out_specs=pl.BlockSpec((BB, C), lambda i: (i, 0)),
        out_shape=jax.ShapeDtypeStruct((Bb, C), jnp.float32),
    )


def kernel(x, embed, W, b):
    B, S = x.shape
    V, D = embed.shape
    C = W.shape[0]
    N = B * S
    idx = x.astype(jnp.int32).reshape(N // _CHUNK, _CHUNK)
    e = _make_sc_gather(V, D, N)(idx, embed)
    e2 = e.reshape(B, S * D)
    head = _make_tc_head(B, S * D, C, 512)
    return head(e2, W.astype(jnp.bfloat16), b.reshape(1, C))


# speedup vs baseline: 24.7253x; 24.7253x over previous
import functools

import jax
import jax.numpy as jnp
from jax import lax
from jax.experimental import pallas as pl
from jax.experimental.pallas import tpu as pltpu
from jax.experimental.pallas import tpu_sc as plsc

_CHUNK = 128
_NBUF = 8


def _make_sc_gather(V, D, N):
    info = plsc.get_sparse_core_info()
    nw = info.num_cores * info.num_subcores
    rows_per_w = N // nw
    n_ch = rows_per_w // _CHUNK
    n_outer = n_ch // _NBUF
    group = _NBUF * _CHUNK
    assert rows_per_w % (_CHUNK * _NBUF) == 0
    mesh = plsc.VectorSubcoreMesh(core_axis_name="c", subcore_axis_name="s")

    @functools.partial(
        pl.kernel,
        mesh=mesh,
        out_type=jax.ShapeDtypeStruct((N, D), jnp.float32),
        scratch_types=[
            pltpu.VMEM((rows_per_w,), jnp.int32),
            pltpu.VMEM((group, D), jnp.float32),
        ] + [pltpu.SemaphoreType.DMA] * _NBUF,
        compiler_params=pltpu.CompilerParams(use_tc_tiling_on_sc=False),
    )
    def gather(idx_hbm, table_hbm, out_hbm, idx_v, rows_v, *sems):
        wid = lax.axis_index("s") * info.num_cores + lax.axis_index("c")
        row_base = wid * rows_per_w
        pltpu.sync_copy(idx_hbm.at[pl.ds(row_base, rows_per_w)], idx_v)

        def body(g, carry):
            ch0 = g * _NBUF
            cps = []
            for j in range(_NBUF):
                cps.append(pltpu.async_copy(
                    table_hbm.at[idx_v.at[pl.ds((ch0 + j) * _CHUNK, _CHUNK)]],
                    rows_v.at[pl.ds(j * _CHUNK, _CHUNK)],
                    sems[j]))
            for cp in cps:
                cp.wait()
            pltpu.sync_copy(
                rows_v,
                out_hbm.at[pl.ds(row_base + ch0 * _CHUNK, group)])
            return carry

        lax.fori_loop(0, n_outer, body, 0)

    return gather


def _make_tc_head(Bb, K, C, BB):
    def body(e_ref, w_ref, b_ref, o_ref):
        e = e_ref[...].astype(jnp.bfloat16)
        logits = lax.dot_general(e, w_ref[...], (((1,), (1,)), ((), ())),
                                 preferred_element_type=jnp.float32)
        logits = logits + b_ref[...]
        m = jnp.max(logits, axis=-1, keepdims=True)
        p = jnp.exp(logits - m)
        o_ref[...] = p / jnp.sum(p, axis=-1, keepdims=True)

    return pl.pallas_call(
        body,
        grid=(Bb // BB,),
        in_specs=[
            pl.BlockSpec((BB, K), lambda i: (i, 0)),
            pl.BlockSpec((C, K), lambda i: (0, 0)),
            pl.BlockSpec((1, C), lambda i: (0, 0)),
        ],
        out_specs=pl.BlockSpec((BB, C), lambda i: (i, 0)),
        out_shape=jax.ShapeDtypeStruct((Bb, C), jnp.float32),
    )


def kernel(x, embed, W, b):
    B, S = x.shape
    V, D = embed.shape
    C = W.shape[0]
    N = B * S
    idx = x.astype(jnp.int32).reshape(N)
    e = _make_sc_gather(V, D, N)(idx, embed)
    e2 = e.reshape(B, S * D)
    head = _make_tc_head(B, S * D, C, 512)
    return head(e2, W.astype(jnp.bfloat16), b.reshape(1, C))
